# Initial kernel scaffold; baseline (speedup 1.0000x reference)
import jax
import jax.numpy as jnp
from jax.experimental import pallas as pl


def kernel(batch_size, position_ids, position_embeddings):
    raise NotImplementedError("write your pallas kernel here")


# speedup vs baseline: 1.3537x; 1.3537x over previous
import functools

import jax
import jax.numpy as jnp
from jax import lax
from jax.experimental import pallas as pl
from jax.experimental.pallas import tpu as pltpu
from jax.experimental.pallas import tpu_sc as plsc

INDEX_DIM = 8192
NUM_CHANNELS = 128
SEQ_LEN = 8192
OUT_BATCH = 4

NUM_CORES = 2
NUM_SUBCORES = 16
NUM_WORKERS = NUM_CORES * NUM_SUBCORES
ROWS_PER_WORKER = SEQ_LEN // NUM_WORKERS
IDX_CHUNK = 128
CHUNKS = ROWS_PER_WORKER // IDX_CHUNK


@functools.partial(
    pl.kernel,
    mesh=plsc.VectorSubcoreMesh(core_axis_name="c", subcore_axis_name="s"),
    out_type=jax.ShapeDtypeStruct((OUT_BATCH, SEQ_LEN, NUM_CHANNELS), jnp.float32),
    scratch_types=[
        pltpu.VMEM((CHUNKS, IDX_CHUNK), jnp.int32),
        pltpu.VMEM((ROWS_PER_WORKER, NUM_CHANNELS), jnp.float32),
        pltpu.SemaphoreType.DMA,
        pltpu.SemaphoreType.DMA,
    ],
)
def _embed_bcast(ids_hbm, table_hbm, out_hbm, idx_v, rows_v, gsem, wsem):
    wid = lax.axis_index("s") * NUM_CORES + lax.axis_index("c")
    base = wid * ROWS_PER_WORKER

    pltpu.sync_copy(ids_hbm.at[pl.ds(wid * CHUNKS, CHUNKS)], idx_v)

    gathers = [
        pltpu.async_copy(
            table_hbm.at[idx_v.at[c]],
            rows_v.at[pl.ds(c * IDX_CHUNK, IDX_CHUNK)],
            gsem,
        )
        for c in range(CHUNKS)
    ]
    for g in gathers:
        g.wait()

    writes = [
        pltpu.async_copy(rows_v, out_hbm.at[b, pl.ds(base, ROWS_PER_WORKER)], wsem)
        for b in range(OUT_BATCH)
    ]
    for w in writes:
        w.wait()


def kernel(batch_size, position_ids, position_embeddings):
    del batch_size
    ids2d = position_ids.reshape(SEQ_LEN // IDX_CHUNK, IDX_CHUNK)
    return _embed_bcast(ids2d, position_embeddings)


# speedup vs baseline: 1.3707x; 1.0125x over previous
import functools

import jax
import jax.numpy as jnp
from jax import lax
from jax.experimental import pallas as pl
from jax.experimental.pallas import tpu as pltpu
from jax.experimental.pallas import tpu_sc as plsc

INDEX_DIM = 8192
NUM_CHANNELS = 128
SEQ_LEN = 8192
OUT_BATCH = 4

NUM_CORES = 2
NUM_SUBCORES = 16
NUM_WORKERS = NUM_CORES * NUM_SUBCORES
ROWS_PER_WORKER = SEQ_LEN // NUM_WORKERS
IDX_CHUNK = 128
CHUNKS = ROWS_PER_WORKER // IDX_CHUNK


@functools.partial(
    pl.kernel,
    mesh=plsc.VectorSubcoreMesh(core_axis_name="c", subcore_axis_name="s"),
    out_type=jax.ShapeDtypeStruct((OUT_BATCH, SEQ_LEN, NUM_CHANNELS), jnp.float32),
    scratch_types=[
        pltpu.VMEM((CHUNKS, IDX_CHUNK), jnp.int32),
        pltpu.VMEM((ROWS_PER_WORKER, NUM_CHANNELS), jnp.float32),
        pltpu.SemaphoreType.DMA,
        pltpu.SemaphoreType.DMA,
    ],
)
def _embed_bcast(ids_hbm, table_hbm, out_hbm, idx_v, rows_v, gsem, wsem):
    wid = lax.axis_index("s") * NUM_CORES + lax.axis_index("c")
    base = wid * ROWS_PER_WORKER

    pltpu.sync_copy(ids_hbm.at[pl.ds(wid * CHUNKS, CHUNKS)], idx_v)

    gathers = [
        pltpu.async_copy(
            table_hbm.at[idx_v.at[c]],
            rows_v.at[pl.ds(c * IDX_CHUNK, IDX_CHUNK)],
            gsem,
        )
        for c in range(CHUNKS)
    ]
    writes = []
    for c in range(CHUNKS):
        gathers[c].wait()
        writes += [
            pltpu.async_copy(
                rows_v.at[pl.ds(c * IDX_CHUNK, IDX_CHUNK)],
                out_hbm.at[b, pl.ds(base + c * IDX_CHUNK, IDX_CHUNK)],
                wsem,
            )
            for b in range(OUT_BATCH)
        ]
    for w in writes:
        w.wait()


def kernel(batch_size, position_ids, position_embeddings):
    del batch_size
    ids2d = position_ids.reshape(SEQ_LEN // IDX_CHUNK, IDX_CHUNK)
    return _embed_bcast(ids2d, position_embeddings)
